# Initial kernel scaffold; baseline (speedup 1.0000x reference)
import jax
import jax.numpy as jnp
from jax.experimental import pallas as pl


def kernel(x, edge_index, W1, b1, W2, b2):
    raise NotImplementedError("write your pallas kernel here")


# speedup vs baseline: 15.0675x; 15.0675x over previous
import functools

import jax
import jax.numpy as jnp
from jax import lax
from jax.experimental import pallas as pl
from jax.experimental.pallas import tpu as pltpu
from jax.experimental.pallas import tpu_sc as plsc

N_NODES = 10000
N_EDGES = 320000
D_FEAT = 128
HIDDEN = 128
N_CLASSES = 16

NPAD = 10240
CHUNK = 100
NCHUNKS = N_EDGES // CHUNK
CPT = NCHUNKS // 32
ROWS_PT = NPAD // 16
ZROWS = 64

BM = 80
GRID = N_NODES // BM

@functools.cache
def _mesh():
    return plsc.VectorSubcoreMesh(
        core_axis_name="c", subcore_axis_name="s", num_cores=2, num_subcores=16
    )


def _sc_degree(ei3, ones_hbm, zeros_hbm):
    @functools.partial(
        pl.kernel,
        out_type=jax.ShapeDtypeStruct((2, NPAD), jnp.float32),
        mesh=_mesh(),
        scratch_types=[
            pltpu.VMEM((2, CHUNK), jnp.int32),
            pltpu.VMEM((CHUNK,), jnp.float32),
            pltpu.VMEM((ROWS_PT,), jnp.float32),
            pltpu.VMEM_SHARED((NPAD,), jnp.float32),
        ],
    )
    def deg_kernel(ei_hbm, ones_h, zeros_h, out_hbm, idx_v, ones_v, stage_v, acc):
        c = lax.axis_index("c")
        s = lax.axis_index("s")
        base = (c * 16 + s) * CPT
        pltpu.sync_copy(ones_h, ones_v)
        pltpu.sync_copy(zeros_h, stage_v)
        pltpu.sync_copy(stage_v, acc.at[pl.ds(s * ROWS_PT, ROWS_PT)])
        plsc.subcore_barrier()

        @pl.loop(0, CPT)
        def _(j):
            pltpu.sync_copy(ei_hbm.at[:, base + j], idx_v)
            pltpu.sync_copy(ones_v, acc.at[idx_v.at[1]], add=True)

        plsc.subcore_barrier()
        pltpu.sync_copy(acc.at[pl.ds(s * ROWS_PT, ROWS_PT)], stage_v)
        pltpu.sync_copy(stage_v, out_hbm.at[c, pl.ds(s * ROWS_PT, ROWS_PT)])

    return deg_kernel(ei3, ones_hbm, zeros_hbm)


def _sc_aggregate(h, ei3, zeros_hbm, width):
    @functools.partial(
        pl.kernel,
        out_type=jax.ShapeDtypeStruct((2, NPAD, width), jnp.float32),
        mesh=_mesh(),
        scratch_types=[
            pltpu.VMEM((2, CHUNK), jnp.int32),
            pltpu.VMEM((2, CHUNK), jnp.int32),
            pltpu.VMEM((CHUNK, width), jnp.float32),
            pltpu.VMEM((CHUNK, width), jnp.float32),
            pltpu.VMEM((ZROWS, width), jnp.float32),
            pltpu.SemaphoreType.DMA,
            pltpu.SemaphoreType.DMA,
            pltpu.VMEM_SHARED((NPAD, width), jnp.float32),
        ],
    )
    def agg_kernel(h_hbm, ei_hbm, z_hbm, out_hbm,
                   idx_a, idx_b, rows_a, rows_b, stage_v, sem_a, sem_b, acc):
        c = lax.axis_index("c")
        s = lax.axis_index("s")
        base = (c * 16 + s) * CPT

        pltpu.sync_copy(z_hbm, stage_v)
        for k in range(ROWS_PT // ZROWS):
            pltpu.sync_copy(stage_v, acc.at[pl.ds(s * ROWS_PT + k * ZROWS, ZROWS)])
        plsc.subcore_barrier()

        def load_idx(buf, j):
            pltpu.sync_copy(ei_hbm.at[:, base + j], buf)

        def gather(idx, rows, sem):
            pltpu.async_copy(h_hbm.at[idx.at[0]], rows, sem)

        def gwait(idx, rows, sem):
            pltpu.make_async_copy(h_hbm.at[idx.at[0]], rows, sem).wait()

        def scat(idx, rows):
            pltpu.sync_copy(rows, acc.at[idx.at[1]], add=True)

        load_idx(idx_a, 0)
        gather(idx_a, rows_a, sem_a)

        @pl.loop(0, CPT // 2 - 1)
        def _(p):
            j = 2 * p
            load_idx(idx_b, j + 1)
            gather(idx_b, rows_b, sem_b)
            gwait(idx_a, rows_a, sem_a)
            scat(idx_a, rows_a)
            load_idx(idx_a, j + 2)
            gather(idx_a, rows_a, sem_a)
            gwait(idx_b, rows_b, sem_b)
            scat(idx_b, rows_b)

        load_idx(idx_b, CPT - 1)
        gather(idx_b, rows_b, sem_b)
        gwait(idx_a, rows_a, sem_a)
        scat(idx_a, rows_a)
        gwait(idx_b, rows_b, sem_b)
        scat(idx_b, rows_b)

        plsc.subcore_barrier()
        for k in range(ROWS_PT // ZROWS):
            r0 = s * ROWS_PT + k * ZROWS
            pltpu.sync_copy(acc.at[pl.ds(r0, ZROWS)], stage_v)
            pltpu.sync_copy(stage_v, out_hbm.at[c, pl.ds(r0, ZROWS)])

    return agg_kernel(h, ei3, zeros_hbm)


def _tc_stage_a(x, W1, degp3):
    def body(x_ref, w_ref, d_ref, o_ref):
        xb = x_ref[...]
        xc = xb - jnp.mean(xb, axis=1, keepdims=True)
        xs = jnp.sign(xc)
        h = jnp.dot(xs, w_ref[...], preferred_element_type=jnp.float32)
        dinv = lax.rsqrt(d_ref[0] + d_ref[1] + 1.0)
        o_ref[...] = dinv * h

    return pl.pallas_call(
        body,
        grid=(GRID,),
        in_specs=[
            pl.BlockSpec((BM, D_FEAT), lambda i: (i, 0)),
            pl.BlockSpec((D_FEAT, HIDDEN), lambda i: (0, 0)),
            pl.BlockSpec((2, BM, 1), lambda i: (0, i, 0)),
        ],
        out_specs=pl.BlockSpec((BM, HIDDEN), lambda i: (i, 0)),
        out_shape=jax.ShapeDtypeStruct((N_NODES, HIDDEN), jnp.float32),
    )(x, W1, degp3)


def _tc_stage_b(S1, h1p, degp3, b1):
    def body(sp_ref, h_ref, d_ref, b_ref, o_ref):
        dinv = lax.rsqrt(d_ref[0] + d_ref[1] + 1.0)
        agg = dinv * (sp_ref[0] + sp_ref[1] + h_ref[...]) + b_ref[...]
        o_ref[...] = dinv * jnp.sign(agg)

    return pl.pallas_call(
        body,
        grid=(GRID,),
        in_specs=[
            pl.BlockSpec((2, BM, HIDDEN), lambda i: (0, i, 0)),
            pl.BlockSpec((BM, HIDDEN), lambda i: (i, 0)),
            pl.BlockSpec((2, BM, 1), lambda i: (0, i, 0)),
            pl.BlockSpec((1, HIDDEN), lambda i: (0, 0)),
        ],
        out_specs=pl.BlockSpec((BM, HIDDEN), lambda i: (i, 0)),
        out_shape=jax.ShapeDtypeStruct((N_NODES, HIDDEN), jnp.float32),
    )(S1, h1p, degp3, b1.reshape(1, HIDDEN))


def _tc_stage_c(S2, s1p, degp3, W2, b2):
    def body(sp_ref, h_ref, d_ref, w_ref, b_ref, o_ref):
        dinv = lax.rsqrt(d_ref[0] + d_ref[1] + 1.0)
        agg = dinv * (sp_ref[0] + sp_ref[1] + h_ref[...])
        z = jnp.dot(agg, w_ref[...], preferred_element_type=jnp.float32)
        z = z + b_ref[...]
        zz = z - jnp.max(z, axis=1, keepdims=True)
        lse = jnp.log(jnp.sum(jnp.exp(zz), axis=1, keepdims=True))
        o_ref[...] = zz - lse

    return pl.pallas_call(
        body,
        grid=(GRID,),
        in_specs=[
            pl.BlockSpec((2, BM, HIDDEN), lambda i: (0, i, 0)),
            pl.BlockSpec((BM, HIDDEN), lambda i: (i, 0)),
            pl.BlockSpec((2, BM, 1), lambda i: (0, i, 0)),
            pl.BlockSpec((HIDDEN, N_CLASSES), lambda i: (0, 0)),
            pl.BlockSpec((1, N_CLASSES), lambda i: (0, 0)),
        ],
        out_specs=pl.BlockSpec((BM, N_CLASSES), lambda i: (i, 0)),
        out_shape=jax.ShapeDtypeStruct((N_NODES, N_CLASSES), jnp.float32),
    )(S2, s1p, degp3, W2, b2.reshape(1, N_CLASSES))


def kernel(x, edge_index, W1, b1, W2, b2):
    ei3 = edge_index.astype(jnp.int32).reshape(2, NCHUNKS, CHUNK)
    ones_e = jnp.ones((CHUNK,), jnp.float32)
    z1 = jnp.zeros((ROWS_PT,), jnp.float32)
    z128 = jnp.zeros((ZROWS, HIDDEN), jnp.float32)

    degp = _sc_degree(ei3, ones_e, z1)
    degp3 = degp.reshape(2, NPAD, 1)
    h1p = _tc_stage_a(x, W1, degp3)
    S1 = _sc_aggregate(h1p, ei3, z128, HIDDEN)
    s1p = _tc_stage_b(S1, h1p, degp3, b1)
    S2 = _sc_aggregate(s1p, ei3, z128, HIDDEN)
    return _tc_stage_c(S2, s1p, degp3, W2, b2)


# speedup vs baseline: 21.6818x; 1.4390x over previous
import functools

import jax
import jax.numpy as jnp
from jax import lax
from jax.experimental import pallas as pl
from jax.experimental.pallas import tpu as pltpu
from jax.experimental.pallas import tpu_sc as plsc

N_NODES = 10000
N_EDGES = 320000
D_FEAT = 128
HIDDEN = 128
N_CLASSES = 16

NPAD = 10240
CHUNK = 100
NCHUNKS = N_EDGES // CHUNK
CPT = NCHUNKS // 32
ROWS_PT = NPAD // 16
ZROWS = 64

BM = 2000
GRID = N_NODES // BM
NBUF = 4

@functools.cache
def _mesh():
    return plsc.VectorSubcoreMesh(
        core_axis_name="c", subcore_axis_name="s", num_cores=2, num_subcores=16
    )


def _sc_degree(ei3, ones_hbm, zeros_hbm):
    @functools.partial(
        pl.kernel,
        out_type=jax.ShapeDtypeStruct((2, NPAD), jnp.float32),
        mesh=_mesh(),
        scratch_types=[
            pltpu.VMEM((CPT, CHUNK), jnp.int32),
            pltpu.VMEM((CHUNK,), jnp.float32),
            [pltpu.SemaphoreType.DMA] * NBUF,
            pltpu.VMEM_SHARED((NPAD,), jnp.float32),
        ],
    )
    def deg_kernel(ei_hbm, ones_h, zeros_h, out_hbm, idx_v, ones_v, sems, acc):
        c = lax.axis_index("c")
        s = lax.axis_index("s")
        pltpu.sync_copy(ones_h, ones_v)
        pltpu.sync_copy(ei_hbm.at[1, c * 16 + s], idx_v)
        pltpu.sync_copy(zeros_h, acc.at[pl.ds(s * ROWS_PT, ROWS_PT)])
        plsc.subcore_barrier()

        def scat(j, sem):
            pltpu.async_copy(ones_v, acc.at[idx_v.at[j]], sem, add=True)

        def swait(j, sem):
            pltpu.make_async_copy(ones_v, acc.at[idx_v.at[j]], sem).wait()

        for b in range(NBUF):
            scat(b, sems[b])

        @pl.loop(0, CPT // NBUF - 1)
        def _(p):
            j = NBUF * p
            for b in range(NBUF):
                swait(j + b, sems[b])
                scat(j + NBUF + b, sems[b])

        for b in range(NBUF):
            swait(CPT - NBUF + b, sems[b])

        plsc.subcore_barrier()
        pltpu.sync_copy(acc.at[pl.ds(s * ROWS_PT, ROWS_PT)],
                        out_hbm.at[c, pl.ds(s * ROWS_PT, ROWS_PT)])

    return deg_kernel(ei3, ones_hbm, zeros_hbm)


def _sc_aggregate(h, ei3, zeros_hbm, width):
    @functools.partial(
        pl.kernel,
        out_type=jax.ShapeDtypeStruct((2, NPAD, width), jnp.float32),
        mesh=_mesh(),
        scratch_types=[
            [pltpu.VMEM((2, CHUNK), jnp.int32)] * 2,
            [pltpu.VMEM((CHUNK, width), jnp.float32)] * 2,
            [pltpu.SemaphoreType.DMA] * 2,
            [pltpu.SemaphoreType.DMA] * 2,
            [pltpu.SemaphoreType.DMA] * 2,
            pltpu.VMEM_SHARED((NPAD, width), jnp.float32),
        ],
    )
    def agg_kernel(h_hbm, ei_hbm, z_hbm, out_hbm,
                   idx, rows, isems, gsems, ssems, acc):
        c = lax.axis_index("c")
        s = lax.axis_index("s")
        w = c * 16 + s

        def iload(j, b):
            pltpu.async_copy(ei_hbm.at[:, w, j], idx[b], isems[b])

        def iwait(j, b):
            pltpu.make_async_copy(ei_hbm.at[:, w, j], idx[b], isems[b]).wait()

        def gather(j, b):
            pltpu.async_copy(h_hbm.at[idx[b].at[0]], rows[b], gsems[b])

        def gwait(j, b):
            pltpu.make_async_copy(h_hbm.at[idx[b].at[0]], rows[b], gsems[b]).wait()

        def scat(j, b):
            pltpu.async_copy(rows[b], acc.at[idx[b].at[1]], ssems[b], add=True)

        def swait(j, b):
            pltpu.make_async_copy(rows[b], acc.at[idx[b].at[1]], ssems[b]).wait()

        iload(0, 0)
        iload(1, 1)
        pltpu.sync_copy(z_hbm, acc.at[pl.ds(s * ROWS_PT, ROWS_PT)])
        plsc.subcore_barrier()
        iwait(0, 0)
        gather(0, 0)
        iwait(1, 1)
        gather(1, 1)

        @pl.loop(0, CPT // 2 - 1)
        def _(p):
            j = 2 * p
            gwait(j, 0)
            scat(j, 0)
            gwait(j + 1, 1)
            scat(j + 1, 1)
            swait(j, 0)
            iload(j + 2, 0)
            swait(j + 1, 1)
            iload(j + 3, 1)
            iwait(j + 2, 0)
            gather(j + 2, 0)
            iwait(j + 3, 1)
            gather(j + 3, 1)

        jl = CPT - 2
        gwait(jl, 0)
        scat(jl, 0)
        gwait(jl + 1, 1)
        scat(jl + 1, 1)
        swait(jl, 0)
        swait(jl + 1, 1)

        plsc.subcore_barrier()
        pltpu.sync_copy(acc.at[pl.ds(s * ROWS_PT, ROWS_PT)],
                        out_hbm.at[c, pl.ds(s * ROWS_PT, ROWS_PT)])

    return agg_kernel(h, ei3, zeros_hbm)


def _tc_stage_a(x, W1, degp3):
    def body(x_ref, w_ref, d_ref, o_ref):
        xb = x_ref[...]
        xc = xb - jnp.mean(xb, axis=1, keepdims=True)
        xs = jnp.sign(xc)
        h = jnp.dot(xs, w_ref[...], preferred_element_type=jnp.float32)
        dinv = lax.rsqrt(d_ref[0] + d_ref[1] + 1.0)
        o_ref[...] = dinv * h

    return pl.pallas_call(
        body,
        grid=(GRID,),
        in_specs=[
            pl.BlockSpec((BM, D_FEAT), lambda i: (i, 0)),
            pl.BlockSpec((D_FEAT, HIDDEN), lambda i: (0, 0)),
            pl.BlockSpec((2, BM, 1), lambda i: (0, i, 0)),
        ],
        out_specs=pl.BlockSpec((BM, HIDDEN), lambda i: (i, 0)),
        out_shape=jax.ShapeDtypeStruct((N_NODES, HIDDEN), jnp.float32),
    )(x, W1, degp3)


def _tc_stage_b(S1, h1p, degp3, b1):
    def body(sp_ref, h_ref, d_ref, b_ref, o_ref):
        dinv = lax.rsqrt(d_ref[0] + d_ref[1] + 1.0)
        agg = dinv * (sp_ref[0] + sp_ref[1] + h_ref[...]) + b_ref[...]
        o_ref[...] = dinv * jnp.sign(agg)

    return pl.pallas_call(
        body,
        grid=(GRID,),
        in_specs=[
            pl.BlockSpec((2, BM, HIDDEN), lambda i: (0, i, 0)),
            pl.BlockSpec((BM, HIDDEN), lambda i: (i, 0)),
            pl.BlockSpec((2, BM, 1), lambda i: (0, i, 0)),
            pl.BlockSpec((1, HIDDEN), lambda i: (0, 0)),
        ],
        out_specs=pl.BlockSpec((BM, HIDDEN), lambda i: (i, 0)),
        out_shape=jax.ShapeDtypeStruct((N_NODES, HIDDEN), jnp.float32),
    )(S1, h1p, degp3, b1.reshape(1, HIDDEN))


def _tc_stage_c(S2, s1p, degp3, W2, b2):
    def body(sp_ref, h_ref, d_ref, w_ref, b_ref, o_ref):
        dinv = lax.rsqrt(d_ref[0] + d_ref[1] + 1.0)
        agg = dinv * (sp_ref[0] + sp_ref[1] + h_ref[...])
        z = jnp.dot(agg, w_ref[...], preferred_element_type=jnp.float32)
        z = z + b_ref[...]
        zz = z - jnp.max(z, axis=1, keepdims=True)
        lse = jnp.log(jnp.sum(jnp.exp(zz), axis=1, keepdims=True))
        o_ref[...] = zz - lse

    return pl.pallas_call(
        body,
        grid=(GRID,),
        in_specs=[
            pl.BlockSpec((2, BM, HIDDEN), lambda i: (0, i, 0)),
            pl.BlockSpec((BM, HIDDEN), lambda i: (i, 0)),
            pl.BlockSpec((2, BM, 1), lambda i: (0, i, 0)),
            pl.BlockSpec((HIDDEN, N_CLASSES), lambda i: (0, 0)),
            pl.BlockSpec((1, N_CLASSES), lambda i: (0, 0)),
        ],
        out_specs=pl.BlockSpec((BM, N_CLASSES), lambda i: (i, 0)),
        out_shape=jax.ShapeDtypeStruct((N_NODES, N_CLASSES), jnp.float32),
    )(S2, s1p, degp3, W2, b2.reshape(1, N_CLASSES))


def kernel(x, edge_index, W1, b1, W2, b2):
    ei3 = edge_index.astype(jnp.int32).reshape(2, 32, CPT, CHUNK)
    ones_e = jnp.ones((CHUNK,), jnp.float32)
    z1 = jnp.zeros((ROWS_PT,), jnp.float32)
    z128 = jnp.zeros((ROWS_PT, HIDDEN), jnp.float32)

    degp = _sc_degree(ei3, ones_e, z1)
    degp3 = degp.reshape(2, NPAD, 1)
    h1p = _tc_stage_a(x, W1, degp3)
    S1 = _sc_aggregate(h1p, ei3, z128, HIDDEN)
    s1p = _tc_stage_b(S1, h1p, degp3, b1)
    S2 = _sc_aggregate(s1p, ei3, z128, HIDDEN)
    return _tc_stage_c(S2, s1p, degp3, W2, b2)


# speedup vs baseline: 28.3754x; 1.3087x over previous
import functools

import jax
import jax.numpy as jnp
from jax import lax
from jax.experimental import pallas as pl
from jax.experimental.pallas import tpu as pltpu
from jax.experimental.pallas import tpu_sc as plsc

N_NODES = 10000
N_EDGES = 320000
D_FEAT = 128
HIDDEN = 128
N_CLASSES = 16

NPAD = 10240
CHUNK = 100
NCHUNKS = N_EDGES // CHUNK
CPT = NCHUNKS // 32
ROWS_PT = NPAD // 16
ZROWS = 64

BM = 2000
GRID = N_NODES // BM
NBUF = 4

@functools.cache
def _mesh():
    return plsc.VectorSubcoreMesh(
        core_axis_name="c", subcore_axis_name="s", num_cores=2, num_subcores=16
    )


def _sc_degree(ei3, ones_hbm, zeros_hbm):
    @functools.partial(
        pl.kernel,
        out_type=jax.ShapeDtypeStruct((2, NPAD), jnp.float32),
        mesh=_mesh(),
        scratch_types=[
            pltpu.VMEM((CPT, CHUNK), jnp.int32),
            pltpu.VMEM((CHUNK,), jnp.float32),
            [pltpu.SemaphoreType.DMA] * NBUF,
            pltpu.VMEM_SHARED((NPAD,), jnp.float32),
        ],
    )
    def deg_kernel(ei_hbm, ones_h, zeros_h, out_hbm, idx_v, ones_v, sems, acc):
        c = lax.axis_index("c")
        s = lax.axis_index("s")
        pltpu.sync_copy(ones_h, ones_v)
        pltpu.sync_copy(ei_hbm.at[1, c * 16 + s], idx_v)
        pltpu.sync_copy(zeros_h, acc.at[pl.ds(s * ROWS_PT, ROWS_PT)])
        plsc.subcore_barrier()

        def scat(j, sem):
            pltpu.async_copy(ones_v, acc.at[idx_v.at[j]], sem, add=True)

        def swait(j, sem):
            pltpu.make_async_copy(ones_v, acc.at[idx_v.at[j]], sem).wait()

        for b in range(NBUF):
            scat(b, sems[b])

        @pl.loop(0, CPT // NBUF - 1)
        def _(p):
            j = NBUF * p
            for b in range(NBUF):
                swait(j + b, sems[b])
                scat(j + NBUF + b, sems[b])

        for b in range(NBUF):
            swait(CPT - NBUF + b, sems[b])

        plsc.subcore_barrier()
        pltpu.sync_copy(acc.at[pl.ds(s * ROWS_PT, ROWS_PT)],
                        out_hbm.at[c, pl.ds(s * ROWS_PT, ROWS_PT)])

    return deg_kernel(ei3, ones_hbm, zeros_hbm)


def _sc_aggregate(h, ei3, zeros_hbm, width):
    @functools.partial(
        pl.kernel,
        out_type=jax.ShapeDtypeStruct((2, NPAD, width), jnp.float32),
        mesh=_mesh(),
        scratch_types=[
            [pltpu.VMEM((1, CHUNK), jnp.int32)] * 3,
            [pltpu.VMEM((1, CHUNK), jnp.int32)] * 3,
            [pltpu.VMEM((CHUNK, width), jnp.float32)] * 3,
            [pltpu.SemaphoreType.DMA] * 3,
            [pltpu.SemaphoreType.DMA] * 3,
            [pltpu.SemaphoreType.DMA] * 3,
            [pltpu.SemaphoreType.DMA] * 3,
            pltpu.VMEM_SHARED((NPAD, width), jnp.float32),
        ],
    )
    def agg_kernel(h_hbm, ei_hbm, z_hbm, out_hbm,
                   sidx, didx, rows, s_isems, d_isems, gsems, ssems, acc):
        c = lax.axis_index("c")
        s = lax.axis_index("s")
        w = c * 16 + s

        def iload_s(j, b):
            pltpu.async_copy(ei_hbm.at[pl.ds(0, 1), w, j], sidx[b], s_isems[b])

        def iwait_s(j, b):
            pltpu.make_async_copy(
                ei_hbm.at[pl.ds(0, 1), w, j], sidx[b], s_isems[b]).wait()

        def iload_d(j, b):
            pltpu.async_copy(ei_hbm.at[pl.ds(1, 1), w, j], didx[b], d_isems[b])

        def iwait_d(j, b):
            pltpu.make_async_copy(
                ei_hbm.at[pl.ds(1, 1), w, j], didx[b], d_isems[b]).wait()

        def gather(j, b):
            pltpu.async_copy(h_hbm.at[sidx[b].at[0]], rows[b], gsems[b])

        def gwait(j, b):
            pltpu.make_async_copy(h_hbm.at[sidx[b].at[0]], rows[b], gsems[b]).wait()

        def scat(j, b):
            pltpu.async_copy(rows[b], acc.at[didx[b].at[0]], ssems[b], add=True)

        def swait(j, b):
            pltpu.make_async_copy(rows[b], acc.at[didx[b].at[0]], ssems[b]).wait()

        for b in range(3):
            iload_s(b, b)
            iload_d(b, b)
        pltpu.sync_copy(z_hbm, acc.at[pl.ds(s * ROWS_PT, ROWS_PT)])
        plsc.subcore_barrier()
        for b in range(3):
            iwait_s(b, b)
            gather(b, b)

        @pl.loop(0, (CPT - 1) // 3)
        def _(p):
            j = 3 * p
            for b in range(3):
                gwait(j + b, b)
                iwait_d(j + b, b)
                scat(j + b, b)

                @pl.when(j + 3 + b < CPT)
                def _():
                    iload_s(j + 3 + b, b)

            for b in range(3):
                swait(j + b, b)

                @pl.when(j + 3 + b < CPT)
                def _():
                    iload_d(j + 3 + b, b)
                    iwait_s(j + 3 + b, b)
                    gather(j + 3 + b, b)

        jl = CPT - 1
        bl = jl % 3
        gwait(jl, bl)
        iwait_d(jl, bl)
        scat(jl, bl)
        swait(jl, bl)

        plsc.subcore_barrier()
        pltpu.sync_copy(acc.at[pl.ds(s * ROWS_PT, ROWS_PT)],
                        out_hbm.at[c, pl.ds(s * ROWS_PT, ROWS_PT)])

    return agg_kernel(h, ei3, zeros_hbm)


def _tc_stage_a(x, W1, degp3):
    def body(x_ref, w_ref, d_ref, o_ref):
        xb = x_ref[...]
        xc = xb - jnp.mean(xb, axis=1, keepdims=True)
        xs = jnp.sign(xc)
        h = jnp.dot(xs, w_ref[...], preferred_element_type=jnp.float32)
        dinv = lax.rsqrt(d_ref[0] + d_ref[1] + 1.0)
        o_ref[...] = dinv * h

    return pl.pallas_call(
        body,
        grid=(GRID,),
        in_specs=[
            pl.BlockSpec((BM, D_FEAT), lambda i: (i, 0)),
            pl.BlockSpec((D_FEAT, HIDDEN), lambda i: (0, 0)),
            pl.BlockSpec((2, BM, 1), lambda i: (0, i, 0)),
        ],
        out_specs=pl.BlockSpec((BM, HIDDEN), lambda i: (i, 0)),
        out_shape=jax.ShapeDtypeStruct((N_NODES, HIDDEN), jnp.float32),
    )(x, W1, degp3)


def _tc_stage_b(S1, h1p, degp3, b1):
    def body(sp_ref, h_ref, d_ref, b_ref, o_ref):
        dinv = lax.rsqrt(d_ref[0] + d_ref[1] + 1.0)
        agg = dinv * (sp_ref[0] + sp_ref[1] + h_ref[...]) + b_ref[...]
        o_ref[...] = dinv * jnp.sign(agg)

    return pl.pallas_call(
        body,
        grid=(GRID,),
        in_specs=[
            pl.BlockSpec((2, BM, HIDDEN), lambda i: (0, i, 0)),
            pl.BlockSpec((BM, HIDDEN), lambda i: (i, 0)),
            pl.BlockSpec((2, BM, 1), lambda i: (0, i, 0)),
            pl.BlockSpec((1, HIDDEN), lambda i: (0, 0)),
        ],
        out_specs=pl.BlockSpec((BM, HIDDEN), lambda i: (i, 0)),
        out_shape=jax.ShapeDtypeStruct((N_NODES, HIDDEN), jnp.float32),
    )(S1, h1p, degp3, b1.reshape(1, HIDDEN))


def _tc_stage_c(S2, s1p, degp3, W2, b2):
    def body(sp_ref, h_ref, d_ref, w_ref, b_ref, o_ref):
        dinv = lax.rsqrt(d_ref[0] + d_ref[1] + 1.0)
        agg = dinv * (sp_ref[0] + sp_ref[1] + h_ref[...])
        z = jnp.dot(agg, w_ref[...], preferred_element_type=jnp.float32)
        z = z + b_ref[...]
        zz = z - jnp.max(z, axis=1, keepdims=True)
        lse = jnp.log(jnp.sum(jnp.exp(zz), axis=1, keepdims=True))
        o_ref[...] = zz - lse

    return pl.pallas_call(
        body,
        grid=(GRID,),
        in_specs=[
            pl.BlockSpec((2, BM, HIDDEN), lambda i: (0, i, 0)),
            pl.BlockSpec((BM, HIDDEN), lambda i: (i, 0)),
            pl.BlockSpec((2, BM, 1), lambda i: (0, i, 0)),
            pl.BlockSpec((HIDDEN, N_CLASSES), lambda i: (0, 0)),
            pl.BlockSpec((1, N_CLASSES), lambda i: (0, 0)),
        ],
        out_specs=pl.BlockSpec((BM, N_CLASSES), lambda i: (i, 0)),
        out_shape=jax.ShapeDtypeStruct((N_NODES, N_CLASSES), jnp.float32),
    )(S2, s1p, degp3, W2, b2.reshape(1, N_CLASSES))


def kernel(x, edge_index, W1, b1, W2, b2):
    ei3 = edge_index.astype(jnp.int32).reshape(2, 32, CPT, CHUNK)
    ones_e = jnp.ones((CHUNK,), jnp.float32)
    z1 = jnp.zeros((ROWS_PT,), jnp.float32)
    z128 = jnp.zeros((ROWS_PT, HIDDEN), jnp.float32)

    degp = _sc_degree(ei3, ones_e, z1)
    degp3 = degp.reshape(2, NPAD, 1)
    h1p = _tc_stage_a(x, W1, degp3)
    S1 = _sc_aggregate(h1p, ei3, z128, HIDDEN)
    s1p = _tc_stage_b(S1, h1p, degp3, b1)
    S2 = _sc_aggregate(s1p, ei3, z128, HIDDEN)
    return _tc_stage_c(S2, s1p, degp3, W2, b2)
